# Initial kernel scaffold; baseline (speedup 1.0000x reference)
import jax
import jax.numpy as jnp
from jax.experimental import pallas as pl


def kernel(pred_logits, pred_boxes, target_sizes):
    raise NotImplementedError("write your pallas kernel here")


# speedup vs baseline: 2.1391x; 2.1391x over previous
import functools

import jax
import jax.numpy as jnp
from jax.experimental import pallas as pl
from jax.experimental.pallas import tpu as pltpu

_K = 100
_RS = 80
_NEG = float("-inf")


def _sigmoid(x):
    pos = x >= 0
    e = jnp.exp(jnp.where(pos, -x, x))
    return jnp.where(pos, 1.0 / (1.0 + e), e / (1.0 + e))


def _body(x_ref, box_ref, scale_ref,
          scores_ref, labels_ref, boxes_ref, prob_ref,
          mval_ref, mrow_ref):
    R = x_ref.shape[1]
    C = x_ref.shape[2]
    S = R // _RS

    def init_slab(s, _):
        xs = x_ref[0, pl.ds(s * _RS, _RS), :]
        mv = jnp.max(xs, axis=0, keepdims=True)
        rows = jax.lax.broadcasted_iota(jnp.int32, (_RS, C), 0) + s * _RS
        mr = jnp.min(jnp.where(xs == mv, rows, jnp.int32(2 ** 30)),
                     axis=0, keepdims=True)
        mval_ref[pl.ds(s, 1), :] = mv
        mrow_ref[pl.ds(s, 1), :] = mr
        return 0

    jax.lax.fori_loop(0, S, init_slab, 0, unroll=False)

    lane128 = jax.lax.broadcasted_iota(jnp.int32, (1, 128), 1)
    lanes_sc = jax.lax.broadcasted_iota(jnp.int32, (S, C), 1)

    def extract(j, carry):
        sc_acc, lb_acc = carry
        mv = mval_ref[...]
        mr = mrow_ref[...]
        vstar = jnp.max(mv)
        flat = mr * C + lanes_sc
        fstar = jnp.min(jnp.where(mv == vstar, flat, jnp.int32(2 ** 31 - 1)))
        rowstar = fstar // C
        lanestar = fstar - rowstar * C
        sstar = rowstar // _RS
        sc_acc = jnp.where(lane128 == j, vstar, sc_acc)
        lb_acc = jnp.where(lane128 == j, lanestar, lb_acc)
        prob_ref[0, pl.ds(j, 1), :] = x_ref[0, pl.ds(rowstar, 1), :]
        boxes_ref[0, pl.ds(j, 1), :] = box_ref[0, pl.ds(rowstar, 1), :]
        xs = x_ref[0, pl.ds(sstar * _RS, _RS), :]
        rows = (jax.lax.broadcasted_iota(jnp.int32, (_RS, C), 0)
                + sstar * _RS)
        flats = rows * C + jax.lax.broadcasted_iota(jnp.int32, (_RS, C), 1)
        keep = (xs < vstar) | ((xs == vstar) & (flats > fstar))
        xm = jnp.where(keep, xs, _NEG)
        mv2 = jnp.max(xm, axis=0, keepdims=True)
        mr2 = jnp.min(jnp.where(xm == mv2, rows, jnp.int32(2 ** 30)),
                      axis=0, keepdims=True)
        mval_ref[pl.ds(sstar, 1), :] = mv2
        mrow_ref[pl.ds(sstar, 1), :] = mr2
        return sc_acc, lb_acc

    z_f = jnp.zeros((1, 128), jnp.float32)
    z_i = jnp.zeros((1, 128), jnp.int32)
    sc_acc, lb_acc = jax.lax.fori_loop(0, _K, extract, (z_f, z_i),
                                       unroll=False)

    scores_ref[0] = _sigmoid(sc_acc)
    labels_ref[0] = lb_acc

    prob_ref[0, pl.ds(0, _K), :] = _sigmoid(prob_ref[0, pl.ds(0, _K), :])

    b = boxes_ref[0, pl.ds(0, _K), :]
    r = jnp.roll(b, 2, axis=1)
    lane4 = jax.lax.broadcasted_iota(jnp.int32, (_K, 4), 1)
    xyxy = jnp.where(lane4 < 2, b - 0.5 * r, r + 0.5 * b)
    boxes_ref[0, pl.ds(0, _K), :] = xyxy * scale_ref[0]


def kernel(pred_logits, pred_boxes, target_sizes):
    B, N, C = pred_logits.shape
    R = ((N + _RS - 1) // _RS) * _RS
    x = jnp.pad(pred_logits, ((0, 0), (0, R - N), (0, 0)),
                constant_values=_NEG)
    bx = jnp.pad(pred_boxes, ((0, 0), (0, R - N), (0, 0)))
    ts = target_sizes.astype(jnp.float32)
    scale = jnp.stack([ts[:, 1], ts[:, 0], ts[:, 1], ts[:, 0]],
                      axis=1).reshape(B, 1, 4)

    S = R // _RS
    grid = (B,)
    out_shapes = [
        jax.ShapeDtypeStruct((B, 1, 128), jnp.float32),
        jax.ShapeDtypeStruct((B, 1, 128), jnp.int32),
        jax.ShapeDtypeStruct((B, 128, 4), jnp.float32),
        jax.ShapeDtypeStruct((B, 128, C), jnp.float32),
    ]
    scores, labels, boxes, prob = pl.pallas_call(
        _body,
        grid=grid,
        in_specs=[
            pl.BlockSpec((1, R, C), lambda b: (b, 0, 0)),
            pl.BlockSpec((1, R, 4), lambda b: (b, 0, 0)),
            pl.BlockSpec((1, 1, 4), lambda b: (b, 0, 0)),
        ],
        out_specs=[
            pl.BlockSpec((1, 1, 128), lambda b: (b, 0, 0)),
            pl.BlockSpec((1, 1, 128), lambda b: (b, 0, 0)),
            pl.BlockSpec((1, 128, 4), lambda b: (b, 0, 0)),
            pl.BlockSpec((1, 128, C), lambda b: (b, 0, 0)),
        ],
        out_shape=out_shapes,
        scratch_shapes=[
            pltpu.VMEM((S, C), jnp.float32),
            pltpu.VMEM((S, C), jnp.int32),
        ],
        compiler_params=pltpu.CompilerParams(
            dimension_semantics=("arbitrary",),
        ),
    )(x, bx, scale)

    return (scores[:, 0, :_K], labels[:, 0, :_K],
            boxes[:, :_K, :], prob[:, :_K, :])


# speedup vs baseline: 2.2989x; 1.0747x over previous
import functools

import jax
import jax.numpy as jnp
from jax.experimental import pallas as pl
from jax.experimental.pallas import tpu as pltpu

_K = 100
_RS = 80
_GB = 8
_NEG = float("-inf")


def _sigmoid(x):
    pos = x >= 0
    e = jnp.exp(jnp.where(pos, -x, x))
    return jnp.where(pos, 1.0 / (1.0 + e), e / (1.0 + e))


def _make_body(gb):
    return functools.partial(_body_impl, gb)


def _body_impl(gb, x_ref,
               scores_ref, labels_ref, rows_ref, prob_ref,
               mval_ref, mrow_ref):
    R = x_ref.shape[1]
    C = x_ref.shape[2]
    S = R // _RS

    def init_slab(s, _):
        for bb in range(gb):
            xs = x_ref[bb, pl.ds(s * _RS, _RS), :]
            mv = jnp.max(xs, axis=0, keepdims=True)
            rows = (jax.lax.broadcasted_iota(jnp.int32, (_RS, C), 0)
                    + s * _RS)
            mr = jnp.min(jnp.where(xs == mv, rows, jnp.int32(2 ** 30)),
                         axis=0, keepdims=True)
            mval_ref[bb, pl.ds(s, 1), :] = mv
            mrow_ref[bb, pl.ds(s, 1), :] = mr
        return 0

    jax.lax.fori_loop(0, S, init_slab, 0, unroll=False)

    lane128 = jax.lax.broadcasted_iota(jnp.int32, (1, 128), 1)

    def extract(j, carry):
        accs = list(carry)
        for bb in range(gb):
            sc_acc, lb_acc, rw_acc = accs[bb]
            mv = mval_ref[bb]
            mr = mrow_ref[bb]
            lanes_sc = jax.lax.broadcasted_iota(jnp.int32, (S, C), 1)
            colmax = jnp.max(mv, axis=0, keepdims=True)
            vstar_v = jnp.max(colmax, axis=1, keepdims=True)
            flat = mr * C + lanes_sc
            cand = jnp.where(mv == vstar_v, flat, jnp.int32(2 ** 31 - 1))
            fstar_v = jnp.min(jnp.min(cand, axis=0, keepdims=True),
                              axis=1, keepdims=True)
            fstar = jnp.min(cand)
            rowstar = fstar // C
            sstar = rowstar // _RS
            rowstar_v = fstar_v // C
            lanestar_v = fstar_v - rowstar_v * C
            sc_acc = jnp.where(lane128 == j,
                               jnp.broadcast_to(vstar_v, (1, 128)), sc_acc)
            lb_acc = jnp.where(lane128 == j,
                               jnp.broadcast_to(lanestar_v, (1, 128)),
                               lb_acc)
            rw_acc = jnp.where(lane128 == j,
                               jnp.broadcast_to(rowstar_v, (1, 128)),
                               rw_acc)
            accs[bb] = (sc_acc, lb_acc, rw_acc)
            prob_ref[bb, pl.ds(j, 1), :] = x_ref[bb, pl.ds(rowstar, 1), :]
            xs = x_ref[bb, pl.ds(sstar * _RS, _RS), :]
            rows = (jax.lax.broadcasted_iota(jnp.int32, (_RS, C), 0)
                    + sstar * _RS)
            flats = (rows * C
                     + jax.lax.broadcasted_iota(jnp.int32, (_RS, C), 1))
            vstar_b = jnp.broadcast_to(vstar_v, (_RS, C))
            fstar_b = jnp.broadcast_to(fstar_v, (_RS, C))
            keep = (xs < vstar_b) | ((xs == vstar_b) & (flats > fstar_b))
            xm = jnp.where(keep, xs, _NEG)
            mv2 = jnp.max(xm, axis=0, keepdims=True)
            mr2 = jnp.min(jnp.where(xm == mv2, rows, jnp.int32(2 ** 30)),
                          axis=0, keepdims=True)
            mval_ref[bb, pl.ds(sstar, 1), :] = mv2
            mrow_ref[bb, pl.ds(sstar, 1), :] = mr2
        return tuple(accs)

    z = tuple((jnp.zeros((1, 128), jnp.float32),
               jnp.zeros((1, 128), jnp.int32),
               jnp.zeros((1, 128), jnp.int32)) for _ in range(gb))
    accs = jax.lax.fori_loop(0, _K, extract, z, unroll=False)

    for bb in range(gb):
        sc_acc, lb_acc, rw_acc = accs[bb]
        scores_ref[bb] = _sigmoid(sc_acc)
        labels_ref[bb] = lb_acc
        rows_ref[bb] = rw_acc
        prob_ref[bb, pl.ds(0, _K), :] = _sigmoid(prob_ref[bb, pl.ds(0, _K), :])


def _box_body(idx_ref, box_ref, scale_ref, out_ref):
    def gather(j, _):
        q = idx_ref[0, 0, j]
        out_ref[0, pl.ds(j, 1), :] = box_ref[0, pl.ds(q, 1), :]
        return 0

    jax.lax.fori_loop(0, _K, gather, 0, unroll=False)
    b = out_ref[0, pl.ds(0, _K), :]
    r = jnp.roll(b, 2, axis=1)
    lane4 = jax.lax.broadcasted_iota(jnp.int32, (_K, 4), 1)
    xyxy = jnp.where(lane4 < 2, b - 0.5 * r, r + 0.5 * b)
    out_ref[0, pl.ds(0, _K), :] = xyxy * scale_ref[0]


def kernel(pred_logits, pred_boxes, target_sizes):
    B, N, C = pred_logits.shape
    R = ((N + _RS - 1) // _RS) * _RS
    x = jnp.pad(pred_logits, ((0, 0), (0, R - N), (0, 0)),
                constant_values=_NEG)
    bx = jnp.pad(pred_boxes, ((0, 0), (0, R - N), (0, 0)))
    ts = target_sizes.astype(jnp.float32)
    scale = jnp.stack([ts[:, 1], ts[:, 0], ts[:, 1], ts[:, 0]],
                      axis=1).reshape(B, 1, 4)

    S = R // _RS
    gb = _GB if B % _GB == 0 else 1
    grid = (B // gb,)
    out_shapes = [
        jax.ShapeDtypeStruct((B, 1, 128), jnp.float32),
        jax.ShapeDtypeStruct((B, 1, 128), jnp.int32),
        jax.ShapeDtypeStruct((B, 1, 128), jnp.int32),
        jax.ShapeDtypeStruct((B, 128, C), jnp.float32),
    ]
    scores, labels, rows, prob = pl.pallas_call(
        _make_body(gb),
        grid=grid,
        in_specs=[
            pl.BlockSpec((gb, R, C), lambda b: (b, 0, 0)),
        ],
        out_specs=[
            pl.BlockSpec((gb, 1, 128), lambda b: (b, 0, 0)),
            pl.BlockSpec((gb, 1, 128), lambda b: (b, 0, 0)),
            pl.BlockSpec((gb, 1, 128), lambda b: (b, 0, 0)),
            pl.BlockSpec((gb, 128, C), lambda b: (b, 0, 0)),
        ],
        out_shape=out_shapes,
        scratch_shapes=[
            pltpu.VMEM((gb, S, C), jnp.float32),
            pltpu.VMEM((gb, S, C), jnp.int32),
        ],
        compiler_params=pltpu.CompilerParams(
            dimension_semantics=("arbitrary",),
        ),
    )(x)

    boxes = pl.pallas_call(
        _box_body,
        grid=(B,),
        in_specs=[
            pl.BlockSpec((1, 1, 128), lambda b: (b, 0, 0),
                         memory_space=pltpu.SMEM),
            pl.BlockSpec((1, R, 4), lambda b: (b, 0, 0)),
            pl.BlockSpec((1, 1, 4), lambda b: (b, 0, 0)),
        ],
        out_specs=pl.BlockSpec((1, 128, 4), lambda b: (b, 0, 0)),
        out_shape=jax.ShapeDtypeStruct((B, 128, 4), jnp.float32),
        compiler_params=pltpu.CompilerParams(
            dimension_semantics=("arbitrary",),
        ),
    )(rows, bx, scale)

    return (scores[:, 0, :_K], labels[:, 0, :_K],
            boxes[:, :_K, :], prob[:, :_K, :])


# speedup vs baseline: 8.1367x; 3.5394x over previous
import functools

import jax
import jax.numpy as jnp
from jax.experimental import pallas as pl
from jax.experimental.pallas import tpu as pltpu

_K = 100
_RS = 40
_GB = 8
_NEG = float("-inf")


def _sigmoid(x):
    pos = x >= 0
    e = jnp.exp(jnp.where(pos, -x, x))
    return jnp.where(pos, 1.0 / (1.0 + e), e / (1.0 + e))


def _make_body(gb):
    return functools.partial(_body_impl, gb)


def _body_impl(gb, x_ref,
               scores_ref, labels_ref, rows_ref, prob_ref,
               *scratch):
    mvals = scratch[:gb]
    mrows = scratch[gb:]
    R = x_ref.shape[1]
    C = x_ref.shape[2]
    S = R // _RS

    def init_slab(s, _):
        for bb in range(gb):
            xs = x_ref[bb, pl.ds(s * _RS, _RS), :]
            mv = jnp.max(xs, axis=0, keepdims=True)
            rows = (jax.lax.broadcasted_iota(jnp.int32, (_RS, C), 0)
                    + s * _RS)
            mr = jnp.min(jnp.where(xs == mv, rows, jnp.int32(2 ** 30)),
                         axis=0, keepdims=True)
            mvals[bb][pl.ds(s, 1), :] = mv
            mrows[bb][pl.ds(s, 1), :] = mr
        return 0

    jax.lax.fori_loop(0, S, init_slab, 0, unroll=False)

    lane128 = jax.lax.broadcasted_iota(jnp.int32, (1, 128), 1)

    def extract(j, carry):
        accs = list(carry)
        for bb in range(gb):
            sc_acc, lb_acc, rw_acc = accs[bb]
            mv = mvals[bb][...]
            mr = mrows[bb][...]
            lanes_sc = jax.lax.broadcasted_iota(jnp.int32, (S, C), 1)
            colmax = jnp.max(mv, axis=0, keepdims=True)
            vstar_v = jnp.max(colmax, axis=1, keepdims=True)
            flat = mr * C + lanes_sc
            cand = jnp.where(mv == vstar_v, flat, jnp.int32(2 ** 31 - 1))
            fstar_v = jnp.min(jnp.min(cand, axis=0, keepdims=True),
                              axis=1, keepdims=True)
            fstar = jnp.min(cand)
            rowstar = fstar // C
            sstar = rowstar // _RS
            rowstar_v = fstar_v // C
            lanestar_v = fstar_v - rowstar_v * C
            sc_acc = jnp.where(lane128 == j,
                               jnp.broadcast_to(vstar_v, (1, 128)), sc_acc)
            lb_acc = jnp.where(lane128 == j,
                               jnp.broadcast_to(lanestar_v, (1, 128)),
                               lb_acc)
            rw_acc = jnp.where(lane128 == j,
                               jnp.broadcast_to(rowstar_v, (1, 128)),
                               rw_acc)
            accs[bb] = (sc_acc, lb_acc, rw_acc)
            prob_ref[bb, pl.ds(j, 1), :] = x_ref[bb, pl.ds(rowstar, 1), :]
            xs = x_ref[bb, pl.ds(sstar * _RS, _RS), :]
            rows = (jax.lax.broadcasted_iota(jnp.int32, (_RS, C), 0)
                    + sstar * _RS)
            flats = (rows * C
                     + jax.lax.broadcasted_iota(jnp.int32, (_RS, C), 1))
            vstar_b = jnp.broadcast_to(vstar_v, (_RS, C))
            fstar_b = jnp.broadcast_to(fstar_v, (_RS, C))
            keep = (xs < vstar_b) | ((xs == vstar_b) & (flats > fstar_b))
            xm = jnp.where(keep, xs, _NEG)
            mv2 = jnp.max(xm, axis=0, keepdims=True)
            mr2 = jnp.min(jnp.where(xm == mv2, rows, jnp.int32(2 ** 30)),
                          axis=0, keepdims=True)
            mvals[bb][pl.ds(sstar, 1), :] = mv2
            mrows[bb][pl.ds(sstar, 1), :] = mr2
        return tuple(accs)

    z = tuple((jnp.zeros((1, 128), jnp.float32),
               jnp.zeros((1, 128), jnp.int32),
               jnp.zeros((1, 128), jnp.int32)) for _ in range(gb))
    accs = jax.lax.fori_loop(0, _K, extract, z, unroll=False)

    for bb in range(gb):
        sc_acc, lb_acc, rw_acc = accs[bb]
        scores_ref[bb] = _sigmoid(sc_acc)
        labels_ref[bb] = lb_acc
        rows_ref[bb] = rw_acc
        prob_ref[bb, pl.ds(0, _K), :] = _sigmoid(prob_ref[bb, pl.ds(0, _K), :])


def _box_body(idx_ref, box_ref, scale_ref, out_ref):
    def gather(j, _):
        q = idx_ref[0, 0, j]
        out_ref[0, pl.ds(j, 1), :] = box_ref[0, pl.ds(q, 1), :]
        return 0

    jax.lax.fori_loop(0, _K, gather, 0, unroll=False)
    b = out_ref[0, pl.ds(0, _K), :]
    r = jnp.roll(b, 2, axis=1)
    lane4 = jax.lax.broadcasted_iota(jnp.int32, (_K, 4), 1)
    xyxy = jnp.where(lane4 < 2, b - 0.5 * r, r + 0.5 * b)
    out_ref[0, pl.ds(0, _K), :] = xyxy * scale_ref[0]


def kernel(pred_logits, pred_boxes, target_sizes):
    B, N, C = pred_logits.shape
    R = ((N + _RS - 1) // _RS) * _RS
    if R == N:
        x = pred_logits
        bx = pred_boxes
    else:
        x = jnp.pad(pred_logits, ((0, 0), (0, R - N), (0, 0)),
                    constant_values=_NEG)
        bx = jnp.pad(pred_boxes, ((0, 0), (0, R - N), (0, 0)))
    ts = target_sizes.astype(jnp.float32)
    scale = jnp.stack([ts[:, 1], ts[:, 0], ts[:, 1], ts[:, 0]],
                      axis=1).reshape(B, 1, 4)

    S = R // _RS
    gb = _GB if B % _GB == 0 else 1
    grid = (B // gb,)
    out_shapes = [
        jax.ShapeDtypeStruct((B, 1, 128), jnp.float32),
        jax.ShapeDtypeStruct((B, 1, 128), jnp.int32),
        jax.ShapeDtypeStruct((B, 1, 128), jnp.int32),
        jax.ShapeDtypeStruct((B, 128, C), jnp.float32),
    ]
    scores, labels, rows, prob = pl.pallas_call(
        _make_body(gb),
        grid=grid,
        in_specs=[
            pl.BlockSpec((gb, R, C), lambda b: (b, 0, 0)),
        ],
        out_specs=[
            pl.BlockSpec((gb, 1, 128), lambda b: (b, 0, 0)),
            pl.BlockSpec((gb, 1, 128), lambda b: (b, 0, 0)),
            pl.BlockSpec((gb, 1, 128), lambda b: (b, 0, 0)),
            pl.BlockSpec((gb, 128, C), lambda b: (b, 0, 0)),
        ],
        out_shape=out_shapes,
        scratch_shapes=(
            [pltpu.VMEM((S, C), jnp.float32) for _ in range(gb)]
            + [pltpu.VMEM((S, C), jnp.int32) for _ in range(gb)]
        ),
        compiler_params=pltpu.CompilerParams(
            dimension_semantics=("arbitrary",),
        ),
    )(x)

    boxes = pl.pallas_call(
        _box_body,
        grid=(B,),
        in_specs=[
            pl.BlockSpec((1, 1, 128), lambda b: (b, 0, 0),
                         memory_space=pltpu.SMEM),
            pl.BlockSpec((1, R, 4), lambda b: (b, 0, 0)),
            pl.BlockSpec((1, 1, 4), lambda b: (b, 0, 0)),
        ],
        out_specs=pl.BlockSpec((1, 128, 4), lambda b: (b, 0, 0)),
        out_shape=jax.ShapeDtypeStruct((B, 128, 4), jnp.float32),
        compiler_params=pltpu.CompilerParams(
            dimension_semantics=("arbitrary",),
        ),
    )(rows, bx, scale)

    return (scores[:, 0, :_K], labels[:, 0, :_K],
            boxes[:, :_K, :], prob[:, :_K, :])
